# baseline (speedup 1.0000x reference)
import jax
import jax.numpy as jnp
from jax.experimental import pallas as pl
from jax.experimental.pallas import tpu as pltpu

_CHUNK = 32
_NSLOTS = 16
_EPS = 1e-8


def _body(x_hbm, amt_ref, out_ref, buf, sems):
    n = x_hbm.shape[0]
    nchunks = n // _CHUNK

    amt = amt_ref[...]
    yn = jnp.sqrt(jnp.sum(amt * amt, axis=0, keepdims=True))

    def copy_in(c, slot):
        return pltpu.make_async_copy(
            x_hbm.at[pl.ds(c * _CHUNK, _CHUNK), :],
            buf.at[slot],
            sems.at[slot],
        )

    for k in range(_NSLOTS):
        copy_in(k, k).start()

    def step(c, carry):
        slot = jax.lax.rem(c, _NSLOTS)
        copy_in(c, slot).wait()
        x = buf[slot]
        dots = jnp.dot(x, amt, preferred_element_type=jnp.float32)
        xn = jnp.sqrt(jnp.sum(x * x, axis=1, keepdims=True))
        den = jnp.maximum(xn * yn, _EPS)
        sims = dots / den
        s0 = sims[:, 0:1]
        s1 = sims[:, 1:2]
        out_ref[pl.ds(c * _CHUNK, _CHUNK), :] = (s1 > s0).astype(jnp.int32)

        @pl.when(c + _NSLOTS < nchunks)
        def _():
            copy_in(c + _NSLOTS, slot).start()

        return carry

    jax.lax.fori_loop(0, nchunks, step, 0)


def kernel(enc_hvs, am):
    n, d = enc_hvs.shape
    amt = am.astype(jnp.float32).T
    out = pl.pallas_call(
        _body,
        in_specs=[
            pl.BlockSpec(memory_space=pltpu.MemorySpace.HBM),
            pl.BlockSpec(memory_space=pltpu.MemorySpace.VMEM),
        ],
        out_specs=pl.BlockSpec(memory_space=pltpu.MemorySpace.VMEM),
        out_shape=jax.ShapeDtypeStruct((n, 1), jnp.int32),
        scratch_shapes=[
            pltpu.VMEM((_NSLOTS, _CHUNK, d), jnp.float32),
            pltpu.SemaphoreType.DMA((_NSLOTS,)),
        ],
        compiler_params=pltpu.CompilerParams(
            vmem_limit_bytes=60 * 1024 * 1024,
        ),
    )(enc_hvs, amt)
    return out.reshape(n)


# speedup vs baseline: 1.1837x; 1.1837x over previous
import jax
import jax.numpy as jnp
from jax.experimental import pallas as pl
from jax.experimental.pallas import tpu as pltpu

_CHUNK = 32
_CPB = 8
_BLOCK = _CHUNK * _CPB
_NBUF = 3
_EPS = 1e-8


def _body(x_hbm, amt_ref, out_ref, buf, sems):
    n = x_hbm.shape[0]
    nblocks = n // _BLOCK

    amt = amt_ref[...]
    yn = jnp.sqrt(jnp.sum(amt * amt, axis=0, keepdims=True))

    def chunk_copy(b, j, slot):
        return pltpu.make_async_copy(
            x_hbm.at[pl.ds(b * _BLOCK + j * _CHUNK, _CHUNK), :],
            buf.at[slot, pl.ds(j * _CHUNK, _CHUNK), :],
            sems.at[slot, j],
        )

    def start_block(b, slot):
        for j in range(_CPB):
            chunk_copy(b, j, slot).start()

    for k in range(_NBUF):
        start_block(k, k)

    def step(b, carry):
        slot = jax.lax.rem(b, _NBUF)
        for j in range(_CPB):
            chunk_copy(b, j, slot).wait()
        x = buf[slot]
        dots = jnp.dot(x, amt, preferred_element_type=jnp.float32)
        xn = jnp.sqrt(jnp.sum(x * x, axis=1, keepdims=True))
        den = jnp.maximum(xn * yn, _EPS)
        sims = dots / den
        s0 = sims[:, 0:1]
        s1 = sims[:, 1:2]
        out_ref[pl.ds(b * _BLOCK, _BLOCK), :] = (s1 > s0).astype(jnp.int32)

        @pl.when(b + _NBUF < nblocks)
        def _():
            start_block(b + _NBUF, slot)

        return carry

    jax.lax.fori_loop(0, nblocks, step, 0)


def kernel(enc_hvs, am):
    n, d = enc_hvs.shape
    amt = am.astype(jnp.float32).T
    out = pl.pallas_call(
        _body,
        in_specs=[
            pl.BlockSpec(memory_space=pltpu.MemorySpace.HBM),
            pl.BlockSpec(memory_space=pltpu.MemorySpace.VMEM),
        ],
        out_specs=pl.BlockSpec(memory_space=pltpu.MemorySpace.VMEM),
        out_shape=jax.ShapeDtypeStruct((n, 1), jnp.int32),
        scratch_shapes=[
            pltpu.VMEM((_NBUF, _BLOCK, d), jnp.float32),
            pltpu.SemaphoreType.DMA((_NBUF, _CPB)),
        ],
        compiler_params=pltpu.CompilerParams(
            vmem_limit_bytes=60 * 1024 * 1024,
        ),
    )(enc_hvs, amt)
    return out.reshape(n)


# speedup vs baseline: 4.7968x; 4.0522x over previous
import jax
import jax.numpy as jnp
from jax.experimental import pallas as pl
from jax.experimental.pallas import tpu as pltpu

_NC = 512
_EPS = 1e-8


def _fused_kernel(x_ref, am_ref, out_ref):
    x = x_ref[...]
    am2 = am_ref[...]
    dots = jax.lax.dot_general(
        am2, x, (((1,), (0,)), ((), ())),
        preferred_element_type=jnp.float32)
    xn = jnp.sqrt(jnp.sum(x * x, axis=0, keepdims=True))
    yn = jnp.sqrt(jnp.sum(am2 * am2, axis=1, keepdims=True))
    den = jnp.maximum(xn * yn, _EPS)
    sims = dots / den
    out_ref[...] = (sims[1:2, :] > sims[0:1, :]).astype(jnp.int32)


def kernel(enc_hvs, am):
    n, d = enc_hvs.shape
    xt = enc_hvs.T
    am = am.astype(jnp.float32)
    out = pl.pallas_call(
        _fused_kernel,
        grid=(n // _NC,),
        in_specs=[
            pl.BlockSpec((d, _NC), lambda i: (0, i)),
            pl.BlockSpec((2, d), lambda i: (0, 0)),
        ],
        out_specs=pl.BlockSpec((1, _NC), lambda i: (0, i)),
        out_shape=jax.ShapeDtypeStruct((1, n), jnp.int32),
        compiler_params=pltpu.CompilerParams(
            dimension_semantics=("arbitrary",),
            vmem_limit_bytes=60 * 1024 * 1024,
        ),
    )(xt, am)
    return out.reshape(n)
